# baseline (speedup 1.0000x reference)
import functools

import jax
import jax.numpy as jnp
from jax import lax
from jax.experimental import pallas as pl
from jax.experimental.pallas import tpu as pltpu
from jax.experimental.pallas import tpu_sc as plsc

_L = 16
_NC = 2
_NS = 16


@functools.cache
def _make_fill(B: int, nc: int = _NC, ns: int = _NS):
    nw = nc * ns
    bpw = B // nw

    mesh = plsc.VectorSubcoreMesh(
        core_axis_name="c", subcore_axis_name="s",
        num_cores=nc, num_subcores=ns,
    )

    @functools.partial(
        pl.kernel,
        out_type=(
            jax.ShapeDtypeStruct((B,), jnp.float32),
            jax.ShapeDtypeStruct((B,), jnp.float32),
        ),
        mesh=mesh,
        compiler_params=pltpu.CompilerParams(needs_layout_passes=False),
        scratch_types=[
            pltpu.VMEM((2,), jnp.float32),
            pltpu.VMEM((bpw,), jnp.float32),
            pltpu.VMEM((bpw,), jnp.float32),
            pltpu.SemaphoreType.DMA,
            pltpu.SemaphoreType.DMA,
        ],
    )
    def fill(param_hbm, out0_hbm, out1_hbm, pv, buf0, buf1, sem0, sem1):
        wid = lax.axis_index("s") * nc + lax.axis_index("c")
        base = wid * bpw
        pltpu.sync_copy(param_hbm, pv)
        v0 = plsc.load_gather(pv, [jnp.zeros((_L,), jnp.int32)])
        v1 = plsc.load_gather(pv, [jnp.ones((_L,), jnp.int32)])
        for i in range(bpw // _L):
            buf0[pl.ds(i * _L, _L)] = v0
        cp0 = pltpu.async_copy(buf0, out0_hbm.at[pl.ds(base, bpw)], sem0)
        for i in range(bpw // _L):
            buf1[pl.ds(i * _L, _L)] = v1
        cp1 = pltpu.async_copy(buf1, out1_hbm.at[pl.ds(base, bpw)], sem1)
        cp0.wait()
        cp1.wait()

    return fill


def kernel(x, x_pa, param):
    B = x.shape[0]
    out0, out1 = _make_fill(B, 1, 1)(param)
    return (out0[:, None], out1[:, None])


# speedup vs baseline: 14.0892x; 14.0892x over previous
import jax
import jax.numpy as jnp
from jax.experimental import pallas as pl
from jax.experimental.pallas import tpu as pltpu


def _fill_body(param_ref, out0_ref, out1_ref):
    out0_ref[...] = jnp.full(out0_ref.shape, param_ref[0], jnp.float32)
    out1_ref[...] = jnp.full(out1_ref.shape, param_ref[1], jnp.float32)


def kernel(x, x_pa, param):
    B = x.shape[0]
    out0, out1 = pl.pallas_call(
        _fill_body,
        out_shape=(
            jax.ShapeDtypeStruct((B // 128, 128), jnp.float32),
            jax.ShapeDtypeStruct((B // 128, 128), jnp.float32),
        ),
        in_specs=[pl.BlockSpec(memory_space=pltpu.SMEM)],
    )(param)
    return (out0.reshape(B, 1), out1.reshape(B, 1))
